# baseline (speedup 1.0000x reference)
import functools

import jax
import jax.numpy as jnp
from jax import lax
from jax.experimental import pallas as pl
from jax.experimental.pallas import tpu as pltpu
from jax.experimental.pallas import tpu_sc as plsc

_ALPHA = 0.2
_NC = 2
_NS = 16
_L = 16
_C = 80
_DW = 144


def _proj_body(x_ref, w_ref, a1_ref, a2_ref, whe_ref, s2_ref):
    wh = jnp.dot(x_ref[...], w_ref[...], preferred_element_type=jnp.float32)
    whe_ref[:, :128] = wh
    whe_ref[:, 128:] = jnp.dot(wh, a1_ref[...],
                               preferred_element_type=jnp.float32)
    s2_ref[...] = jnp.dot(wh, a2_ref[...], preferred_element_type=jnp.float32)


def _final_body(n_rows, acc_ref, r_ref, out_ref):
    acc = acc_ref[0][:n_rows] + acc_ref[1][:n_rows]
    num = acc[:, :128]
    den = acc[:, 128:]
    dcols = jnp.dot(den, r_ref[...], preferred_element_type=jnp.float32)
    v = num / (dcols + 1e-16)
    out_ref[...] = jnp.where(v > 0, v, jnp.exp(v) - 1.0)


def _sc_edge_kernel(N, NP, n_chunks, per_tile):
    rows_per_tile = NP // _NS
    n_cp = rows_per_tile // _C
    mesh = plsc.VectorSubcoreMesh(core_axis_name="c", subcore_axis_name="s")

    @functools.partial(
        pl.kernel,
        out_type=jax.ShapeDtypeStruct((_NC, NP, _DW), jnp.float32),
        mesh=mesh,
        compiler_params=pltpu.CompilerParams(
            use_tc_tiling_on_sc=False, needs_layout_passes=False),
        scratch_types=[
            [pltpu.VMEM((_C,), jnp.int32) for _ in range(3)],
            [pltpu.VMEM((_C,), jnp.int32) for _ in range(3)],
            [pltpu.VMEM((_C, _DW), jnp.float32) for _ in range(3)],
            [pltpu.VMEM((_C, 8), jnp.float32) for _ in range(3)],
            [pltpu.SemaphoreType.DMA for _ in range(3)],
            [pltpu.SemaphoreType.DMA for _ in range(3)],
            pltpu.VMEM_SHARED((NP, _DW), jnp.float32),
        ],
    )
    def body(whe_hbm, s2_hbm, src_hbm, dst_hbm, acc_out,
             src_v, dst_v, whs_v, sd_v, gsem, ssem, acc_sh):
        cid = lax.axis_index("c")
        sid = lax.axis_index("s")
        wid = cid * _NS + sid
        base = sid * rows_per_tile
        ebase = wid * per_tile
        zero16 = jnp.zeros((_L,), jnp.float32)
        lanes = lax.iota(jnp.int32, _L)

        def _zrow(r, _):
            for b in range(_DW // _L):
                whs_v[0][r, pl.ds(b * _L, _L)] = zero16
            return 0
        lax.fori_loop(0, _C, _zrow, 0)
        for r in range(n_cp):
            pltpu.async_copy(whs_v[0], acc_sh.at[pl.ds(base + r * _C, _C)],
                             gsem[0])
        for r in range(n_cp):
            pltpu.make_async_copy(whs_v[0],
                                  acc_sh.at[pl.ds(base + r * _C, _C)],
                                  gsem[0]).wait()
        plsc.subcore_barrier()

        def _idx_and_gather(j, k):
            pltpu.sync_copy(src_hbm.at[pl.ds(ebase + j * _C, _C)], src_v[k])
            pltpu.sync_copy(dst_hbm.at[pl.ds(ebase + j * _C, _C)], dst_v[k])
            pltpu.async_copy(whe_hbm.at[src_v[k]], whs_v[k], gsem[k])
            pltpu.async_copy(s2_hbm.at[dst_v[k]], sd_v[k], gsem[k])

        def _wait_scatter(k):
            pltpu.make_async_copy(whs_v[k], acc_sh.at[dst_v[k]],
                                  ssem[k]).wait()

        def _step(j, k, prefetch, first):
            pltpu.make_async_copy(whe_hbm.at[src_v[k]], whs_v[k],
                                  gsem[k]).wait()
            pltpu.make_async_copy(s2_hbm.at[dst_v[k]], sd_v[k],
                                  gsem[k]).wait()
            for g in range(_C // _L):
                rows = lanes + g * _L
                for h in range(4):
                    col = jnp.full((_L,), 128 + h, jnp.int32)
                    s1 = plsc.load_gather(whs_v[k], [rows, col])
                    s2 = plsc.load_gather(
                        sd_v[k], [rows, jnp.full((_L,), h, jnp.int32)])
                    e = s1 + s2
                    e = jnp.maximum(e, _ALPHA * e)
                    plsc.store_scatter(whs_v[k], [rows, col], jnp.exp(e))

            @plsc.parallel_loop(0, _C, step=1, unroll=8)
            def _scale(i):
                exv = whs_v[k][i, pl.ds(128, _L)]
                for h in range(4):
                    exs = exv[h]
                    for b in range(2):
                        c0 = h * 32 + b * _L
                        whs_v[k][i, pl.ds(c0, _L)] = (
                            whs_v[k][i, pl.ds(c0, _L)] * exs)

            pltpu.async_copy(whs_v[k], acc_sh.at[dst_v[k]], ssem[k], add=True)
            kp2 = (k + 2) % 3
            if prefetch:
                if first:
                    _idx_and_gather(j + 2, kp2)
                else:
                    _wait_scatter(kp2)
                    _idx_and_gather(j + 2, kp2)

        _idx_and_gather(0, 0)
        _idx_and_gather(1, 1)

        def _loop(p, _):
            j = 3 * p
            _step(j, 0, True, False)
            _step(j + 1, 1, True, False)
            _step(j + 2, 2, True, False)
            return 0
        _step(0, 0, True, True)
        _step(1, 1, True, False)
        _step(2, 2, True, False)
        lax.fori_loop(1, n_chunks // 3, _loop, 0)
        _step(n_chunks - 2, 0, False, False)
        _step(n_chunks - 1, 1, False, False)
        _wait_scatter(0)
        _wait_scatter(1)
        _wait_scatter(2)
        plsc.subcore_barrier()

        for r in range(n_cp):
            k = r % 3
            if r >= 3:
                pltpu.make_async_copy(
                    whs_v[k],
                    acc_out.at[cid, pl.ds(base + (r - 3) * _C, _C)],
                    ssem[k]).wait()
            pltpu.async_copy(acc_sh.at[pl.ds(base + r * _C, _C)], whs_v[k],
                             gsem[k]).wait()
            pltpu.async_copy(whs_v[k],
                             acc_out.at[cid, pl.ds(base + r * _C, _C)],
                             ssem[k])
        for r in range(n_cp - 3, n_cp):
            k = r % 3
            pltpu.make_async_copy(
                whs_v[k], acc_out.at[cid, pl.ds(base + r * _C, _C)],
                ssem[k]).wait()

    return body


def kernel(x, edge_index, W, a):
    N, D_IN = x.shape
    E = edge_index.shape[1]
    H, _, D_OUT = W.shape
    DCAT = H * D_OUT

    w_cat = jnp.transpose(W, (1, 0, 2)).reshape(D_IN, DCAT)
    eye = jnp.eye(H, dtype=jnp.float32)
    a1 = (a[:, :D_OUT][:, :, None] * eye[:, None, :]).reshape(DCAT, H)
    a2 = (a[:, D_OUT:][:, :, None] * eye[:, None, :]).reshape(DCAT, H)
    a1_ext = jnp.pad(a1, ((0, 0), (0, _DW - DCAT - H)))
    a2_ext = jnp.pad(a2, ((0, 0), (0, 4)))
    cols = jnp.arange(DCAT, dtype=jnp.int32) // D_OUT
    r_mat = (jnp.arange(_DW - DCAT, dtype=jnp.int32)[:, None]
             == cols[None, :]).astype(jnp.float32)

    n_tiles = _NC * _NS
    per_tile = E // n_tiles
    n_chunks = per_tile // _C
    NP = ((N + _C * _NS - 1) // (_C * _NS)) * (_C * _NS)
    src1 = edge_index[0]
    dst1 = edge_index[1]

    wh_ext, s2_tab = pl.pallas_call(
        _proj_body,
        out_shape=(
            jax.ShapeDtypeStruct((N, _DW), jnp.float32),
            jax.ShapeDtypeStruct((N, 8), jnp.float32),
        ),
    )(x, w_cat, a1_ext, a2_ext)

    acc = _sc_edge_kernel(N, NP, n_chunks, per_tile)(
        wh_ext, s2_tab, src1, dst1)

    out = pl.pallas_call(
        functools.partial(_final_body, N),
        out_shape=jax.ShapeDtypeStruct((N, DCAT), jnp.float32),
    )(acc, r_mat)
    return out


# speedup vs baseline: 1.0997x; 1.0997x over previous
import functools

import jax
import jax.numpy as jnp
from jax import lax
from jax.experimental import pallas as pl
from jax.experimental.pallas import tpu as pltpu
from jax.experimental.pallas import tpu_sc as plsc

_ALPHA = 0.2
_NC = 2
_NS = 16
_L = 16
_C = 80
_DW = 144


def _proj_body(x_ref, w_ref, a1_ref, a2_ref, whe_ref, s2_ref):
    wh = jnp.dot(x_ref[...], w_ref[...], preferred_element_type=jnp.float32)
    whe_ref[:, :128] = wh
    whe_ref[:, 128:] = jnp.dot(wh, a1_ref[...],
                               preferred_element_type=jnp.float32)
    s2_ref[...] = jnp.dot(wh, a2_ref[...], preferred_element_type=jnp.float32)


def _final_body(n_rows, acc_ref, r_ref, out_ref):
    acc = acc_ref[0][:n_rows] + acc_ref[1][:n_rows]
    num = acc[:, :128]
    den = acc[:, 128:]
    dcols = jnp.dot(den, r_ref[...], preferred_element_type=jnp.float32)
    v = num / (dcols + 1e-16)
    out_ref[...] = jnp.where(v > 0, v, jnp.exp(v) - 1.0)


def _sc_edge_kernel(N, NP, n_chunks, per_tile):
    rows_per_tile = NP // _NS
    n_cp = rows_per_tile // _C
    mesh = plsc.VectorSubcoreMesh(core_axis_name="c", subcore_axis_name="s")

    @functools.partial(
        pl.kernel,
        out_type=jax.ShapeDtypeStruct((_NC, NP, _DW), jnp.float32),
        mesh=mesh,
        compiler_params=pltpu.CompilerParams(
            use_tc_tiling_on_sc=False, needs_layout_passes=False),
        scratch_types=[
            [pltpu.VMEM((2, _C), jnp.int32) for _ in range(3)],
            [pltpu.VMEM((_C, _DW), jnp.float32) for _ in range(3)],
            [pltpu.VMEM((_C, 8), jnp.float32) for _ in range(3)],
            [pltpu.SemaphoreType.DMA for _ in range(3)],
            [pltpu.SemaphoreType.DMA for _ in range(3)],
            pltpu.VMEM_SHARED((NP, _DW), jnp.float32),
        ],
    )
    def body(whe_hbm, s2_hbm, ed_hbm, acc_out,
             ed_v, whs_v, sd_v, gsem, ssem, acc_sh):
        cid = lax.axis_index("c")
        sid = lax.axis_index("s")
        wid = cid * _NS + sid
        base = sid * rows_per_tile
        ebase = wid * per_tile
        zero16 = jnp.zeros((_L,), jnp.float32)
        lanes = lax.iota(jnp.int32, _L)

        def _zrow(r, _):
            for b in range(_DW // _L):
                whs_v[0][r, pl.ds(b * _L, _L)] = zero16
            return 0
        lax.fori_loop(0, _C, _zrow, 0)
        for r in range(n_cp):
            pltpu.async_copy(whs_v[0], acc_sh.at[pl.ds(base + r * _C, _C)],
                             gsem[0])
        for r in range(n_cp):
            pltpu.make_async_copy(whs_v[0],
                                  acc_sh.at[pl.ds(base + r * _C, _C)],
                                  gsem[0]).wait()
        plsc.subcore_barrier()

        def _idx_and_gather(j, k):
            pltpu.sync_copy(ed_hbm.at[wid * n_chunks + j], ed_v[k])
            pltpu.async_copy(whe_hbm.at[ed_v[k].at[0]], whs_v[k], gsem[k])
            pltpu.async_copy(s2_hbm.at[ed_v[k].at[1]], sd_v[k], gsem[k])

        def _wait_scatter(k):
            pltpu.make_async_copy(whs_v[k], acc_sh.at[ed_v[k].at[1]],
                                  ssem[k]).wait()

        def _step(j, k, prefetch, first):
            pltpu.make_async_copy(whe_hbm.at[ed_v[k].at[0]], whs_v[k],
                                  gsem[k]).wait()
            pltpu.make_async_copy(s2_hbm.at[ed_v[k].at[1]], sd_v[k],
                                  gsem[k]).wait()
            for g in range(_C // _L):
                rows = lanes + g * _L
                for h in range(4):
                    col = jnp.full((_L,), 128 + h, jnp.int32)
                    s1 = plsc.load_gather(whs_v[k], [rows, col])
                    s2 = plsc.load_gather(
                        sd_v[k], [rows, jnp.full((_L,), h, jnp.int32)])
                    e = s1 + s2
                    e = jnp.maximum(e, _ALPHA * e)
                    plsc.store_scatter(whs_v[k], [rows, col], jnp.exp(e))

            @plsc.parallel_loop(0, _C, step=1, unroll=8)
            def _scale(i):
                exv = whs_v[k][i, pl.ds(128, _L)]
                for h in range(4):
                    exs = exv[h]
                    for b in range(2):
                        c0 = h * 32 + b * _L
                        whs_v[k][i, pl.ds(c0, _L)] = (
                            whs_v[k][i, pl.ds(c0, _L)] * exs)

            pltpu.async_copy(whs_v[k], acc_sh.at[ed_v[k].at[1]], ssem[k],
                             add=True)
            kp2 = (k + 2) % 3
            if prefetch:
                if first:
                    _idx_and_gather(j + 2, kp2)
                else:
                    _wait_scatter(kp2)
                    _idx_and_gather(j + 2, kp2)

        _idx_and_gather(0, 0)
        _idx_and_gather(1, 1)

        def _loop(p, _):
            j = 3 * p
            _step(j, 0, True, False)
            _step(j + 1, 1, True, False)
            _step(j + 2, 2, True, False)
            return 0
        _step(0, 0, True, True)
        _step(1, 1, True, False)
        _step(2, 2, True, False)
        lax.fori_loop(1, n_chunks // 3, _loop, 0)
        _step(n_chunks - 2, 0, False, False)
        _step(n_chunks - 1, 1, False, False)
        _wait_scatter(0)
        _wait_scatter(1)
        _wait_scatter(2)
        plsc.subcore_barrier()

        for r in range(n_cp):
            k = r % 3
            if r >= 3:
                pltpu.make_async_copy(
                    whs_v[k],
                    acc_out.at[cid, pl.ds(base + (r - 3) * _C, _C)],
                    ssem[k]).wait()
            pltpu.async_copy(acc_sh.at[pl.ds(base + r * _C, _C)], whs_v[k],
                             gsem[k]).wait()
            pltpu.async_copy(whs_v[k],
                             acc_out.at[cid, pl.ds(base + r * _C, _C)],
                             ssem[k])
        for r in range(n_cp - 3, n_cp):
            k = r % 3
            pltpu.make_async_copy(
                whs_v[k], acc_out.at[cid, pl.ds(base + r * _C, _C)],
                ssem[k]).wait()

    return body


def kernel(x, edge_index, W, a):
    N, D_IN = x.shape
    E = edge_index.shape[1]
    H, _, D_OUT = W.shape
    DCAT = H * D_OUT

    w_cat = jnp.transpose(W, (1, 0, 2)).reshape(D_IN, DCAT)
    eye = jnp.eye(H, dtype=jnp.float32)
    a1 = (a[:, :D_OUT][:, :, None] * eye[:, None, :]).reshape(DCAT, H)
    a2 = (a[:, D_OUT:][:, :, None] * eye[:, None, :]).reshape(DCAT, H)
    a1_ext = jnp.pad(a1, ((0, 0), (0, _DW - DCAT - H)))
    a2_ext = jnp.pad(a2, ((0, 0), (0, 4)))
    cols = jnp.arange(DCAT, dtype=jnp.int32) // D_OUT
    r_mat = (jnp.arange(_DW - DCAT, dtype=jnp.int32)[:, None]
             == cols[None, :]).astype(jnp.float32)

    n_tiles = _NC * _NS
    per_tile = E // n_tiles
    n_chunks = per_tile // _C
    NP = ((N + _C * _NS - 1) // (_C * _NS)) * (_C * _NS)

    wh_ext, s2_tab = pl.pallas_call(
        _proj_body,
        out_shape=(
            jax.ShapeDtypeStruct((N, _DW), jnp.float32),
            jax.ShapeDtypeStruct((N, 8), jnp.float32),
        ),
    )(x, w_cat, a1_ext, a2_ext)

    ed_pk = jnp.stack([edge_index[0].reshape(n_tiles * n_chunks, _C),
                       edge_index[1].reshape(n_tiles * n_chunks, _C)],
                      axis=1)
    acc = _sc_edge_kernel(N, NP, n_chunks, per_tile)(
        wh_ext, s2_tab, ed_pk)

    out = pl.pallas_call(
        functools.partial(_final_body, N),
        out_shape=jax.ShapeDtypeStruct((N, DCAT), jnp.float32),
    )(acc, r_mat)
    return out
